# baseline (speedup 1.0000x reference)
import functools

import jax
import jax.numpy as jnp
from jax import lax
from jax.experimental import pallas as pl
from jax.experimental.pallas import tpu as pltpu
from jax.experimental.pallas import tpu_sc as plsc

_F32 = jnp.float32
_I32 = jnp.int32


def _segment_sum_sc(table2, edges, n_nodes, n_edges):
    E = n_edges
    NS = 16
    ept = E // NS
    assert ept * NS == E
    nchunk, tail = divmod(ept, 128)
    assert tail % 16 == 0
    WB = 40
    RPT = 1000
    n_wtiles = n_nodes // RPT
    assert n_wtiles * RPT == n_nodes and n_wtiles <= NS and RPT % WB == 0
    nwr = RPT // WB

    mesh = plsc.VectorSubcoreMesh(core_axis_name="c", subcore_axis_name="s")

    assert nchunk % 2 == 0 and nchunk >= 4
    SCW = 1024
    assert tail == 0 or nchunk % 8 != 0
    _dmax = max(ept * (NS - 1) + (nchunk - nchunk % 8) * 128 - (E - SCW), 0)
    assert _dmax % 16 == 0 and _dmax + (nchunk % 8) * 128 + tail <= SCW

    scratch = [
        pltpu.VMEM((SCW,), _I32),
        pltpu.VMEM((SCW,), _I32),
        pltpu.VMEM((128,), _I32),
        pltpu.VMEM((128,), _I32),
        pltpu.VMEM((128,), _I32),
        pltpu.VMEM((128,), _I32),
        pltpu.VMEM((128, 128), _F32),
        pltpu.VMEM((128, 128), _F32),
        pltpu.VMEM((WB, 128), _F32),
        pltpu.VMEM_SHARED((n_nodes, 128), _F32),
        pltpu.SemaphoreType.DMA,
        pltpu.SemaphoreType.DMA,
        pltpu.SemaphoreType.DMA,
        pltpu.SemaphoreType.DMA,
    ]
    if tail:
        scratch += [
            pltpu.VMEM((tail,), _I32),
            pltpu.VMEM((tail,), _I32),
            pltpu.VMEM((tail, 128), _F32),
        ]

    @functools.partial(
        pl.kernel,
        mesh=mesh,
        out_type=jax.ShapeDtypeStruct((2, n_nodes, 128), _F32),
        scratch_types=scratch,
    )
    def k(table_hbm, edges_hbm, out_hbm,
          srcS, dstS, idxA, dstA, idxB, dstB, rowsA, rowsB,
          zbuf, acc, semA, semB, semSA, semSB, *tails):
        c = lax.axis_index("c")
        s = lax.axis_index("s")

        def zrow(r, carry):
            for j in range(8):
                zbuf[r, pl.ds(j * 16, 16)] = jnp.zeros((16,), _F32)
            return carry
        lax.fori_loop(0, WB, zrow, 0)

        row0 = s * RPT

        @pl.when(s < n_wtiles)
        def _zero():
            def zcp(t, carry):
                pltpu.sync_copy(zbuf, acc.at[pl.ds(row0 + t * WB, WB)])
                return carry
            lax.fori_loop(0, nwr, zcp, 0)

        ebase = s * ept
        plsc.subcore_barrier()

        def fill(cidx, idx_v, dst_v):
            b0 = ebase + (cidx - cidx % 8) * 128
            d = jnp.maximum(b0 - (E - SCW), 0)

            @pl.when(cidx % 8 == 0)
            def _stage():
                bs = pl.multiple_of(b0 - d, 8)
                pltpu.sync_copy(edges_hbm.at[pl.ds(bs, SCW)], srcS)
                pltpu.sync_copy(edges_hbm.at[pl.ds(E + bs, SCW)], dstS)
            loc = d + (cidx % 8) * 128
            for j in range(8):
                sv = srcS[pl.ds(loc + j * 16, 16)]
                idx_v[pl.ds(j * 16, 16)] = sv * 2 + c
                dst_v[pl.ds(j * 16, 16)] = dstS[pl.ds(loc + j * 16, 16)]

        fill(jnp.int32(0), idxA, dstA)
        pltpu.async_copy(table_hbm.at[idxA], rowsA, semA)
        fill(jnp.int32(1), idxB, dstB)
        pltpu.async_copy(table_hbm.at[idxB], rowsB, semB)

        def step(i2, carry):
            pltpu.make_async_copy(table_hbm.at[idxA], rowsA, semA).wait()
            pltpu.async_copy(rowsA, acc.at[dstA], semSA, add=True)
            pltpu.make_async_copy(table_hbm.at[idxB], rowsB, semB).wait()
            pltpu.async_copy(rowsB, acc.at[dstB], semSB, add=True)
            pltpu.make_async_copy(rowsA, acc.at[dstA], semSA).wait()
            fill(2 * i2 + 2, idxA, dstA)
            pltpu.async_copy(table_hbm.at[idxA], rowsA, semA)
            pltpu.make_async_copy(rowsB, acc.at[dstB], semSB).wait()
            fill(2 * i2 + 3, idxB, dstB)
            pltpu.async_copy(table_hbm.at[idxB], rowsB, semB)
            return carry
        lax.fori_loop(0, nchunk // 2 - 1, step, 0)

        pltpu.make_async_copy(table_hbm.at[idxA], rowsA, semA).wait()
        pltpu.async_copy(rowsA, acc.at[dstA], semSA, add=True)
        pltpu.make_async_copy(table_hbm.at[idxB], rowsB, semB).wait()
        pltpu.async_copy(rowsB, acc.at[dstB], semSB, add=True)
        pltpu.make_async_copy(rowsA, acc.at[dstA], semSA).wait()
        pltpu.make_async_copy(rowsB, acc.at[dstB], semSB).wait()

        if tail:
            idx_t, dst_t, rows_t = tails
            b0t = ebase + (nchunk - nchunk % 8) * 128
            off = jnp.maximum(b0t - (E - SCW), 0) + (nchunk % 8) * 128
            for j in range(tail // 16):
                sv = srcS[pl.ds(off + j * 16, 16)]
                idx_t[pl.ds(j * 16, 16)] = sv * 2 + c
                dst_t[pl.ds(j * 16, 16)] = dstS[pl.ds(off + j * 16, 16)]
            pltpu.async_copy(table_hbm.at[idx_t], rows_t, semA).wait()
            pltpu.sync_copy(rows_t, acc.at[dst_t], add=True)

        plsc.subcore_barrier()

        @pl.when(s < n_wtiles)
        def _writeout():
            def wout(t, carry):
                r = row0 + t * WB
                pltpu.sync_copy(acc.at[pl.ds(r, WB)], zbuf)
                pltpu.sync_copy(zbuf, out_hbm.at[c, pl.ds(r, WB)])
                return carry
            lax.fori_loop(0, nwr, wout, 0)

    return k(table2, edges)


def _gin_mlp_tc(x, aggT, Wa, ba, Wb, bb, relu_out, interleave_out):
    N, D = x.shape
    BN = 1000
    assert N % BN == 0
    grid = (N // BN,)
    dn = (((1,), (0,)), ((), ()))

    def body(x_ref, agg_ref, wa_ref, ba_ref, wb_ref, bb_ref, o_ref):
        h = x_ref[...] + jnp.concatenate([agg_ref[0], agg_ref[1]], axis=-1)
        u = lax.dot_general(h, wa_ref[...], dn,
                            preferred_element_type=_F32)
        u = jnp.maximum(u + ba_ref[...], 0.0)
        y = lax.dot_general(u, wb_ref[...], dn,
                            preferred_element_type=_F32)
        y = y + bb_ref[...]
        if relu_out:
            y = jnp.maximum(y, 0.0)
        o_ref[...] = y.reshape(o_ref.shape)

    if interleave_out:
        out_sds = jax.ShapeDtypeStruct((N, 2, 128), _F32)
        out_spec = pl.BlockSpec((BN, 2, 128), lambda i: (i, 0, 0))
    else:
        out_sds = jax.ShapeDtypeStruct((N, D), _F32)
        out_spec = pl.BlockSpec((BN, D), lambda i: (i, 0))

    return pl.pallas_call(
        body,
        grid=grid,
        in_specs=[
            pl.BlockSpec((BN, D), lambda i: (i, 0)),
            pl.BlockSpec((2, BN, 128), lambda i: (0, i, 0)),
            pl.BlockSpec((D, D), lambda i: (0, 0)),
            pl.BlockSpec((1, D), lambda i: (0, 0)),
            pl.BlockSpec((D, D), lambda i: (0, 0)),
            pl.BlockSpec((1, D), lambda i: (0, 0)),
        ],
        out_specs=out_spec,
        out_shape=out_sds,
    )(x, aggT, Wa, ba.reshape(1, D), Wb, bb.reshape(1, D))


def kernel(x, ei, W1a, b1a, W1b, b1b, W2a, b2a, W2b, b2b):
    N, D = x.shape
    assert D == 256
    ei32 = ei.astype(_I32)

    E = ei.shape[1]
    x2 = x.reshape(2 * N, 128)
    eiflat = ei32.reshape(2 * E)
    agg1 = _segment_sum_sc(x2, eiflat, N, E)
    y1i = _gin_mlp_tc(x, agg1, W1a, b1a, W1b, b1b,
                      relu_out=True, interleave_out=True)
    agg2 = _segment_sum_sc(y1i.reshape(2 * N, 128), eiflat, N, E)
    out = _gin_mlp_tc(y1i.reshape(N, D), agg2, W2a, b2a, W2b, b2b,
                      relu_out=False, interleave_out=False)
    return out


# speedup vs baseline: 1.2454x; 1.2454x over previous
import functools

import jax
import jax.numpy as jnp
from jax import lax
from jax.experimental import pallas as pl
from jax.experimental.pallas import tpu as pltpu
from jax.experimental.pallas import tpu_sc as plsc

_F32 = jnp.float32
_I32 = jnp.int32


def _segment_sum_sc(table2, edges, n_nodes, n_edges):
    E = n_edges
    NS = 16
    ept = E // NS
    assert ept * NS == E
    nchunk, tail = divmod(ept, 128)
    assert tail % 16 == 0
    WB = 40
    RPT = 1000
    n_wtiles = n_nodes // RPT
    assert n_wtiles * RPT == n_nodes and n_wtiles <= NS and RPT % WB == 0
    nwr = RPT // WB

    mesh = plsc.VectorSubcoreMesh(core_axis_name="c", subcore_axis_name="s")

    assert nchunk % 2 == 0 and nchunk >= 4
    SCW = 1024
    assert tail == 0 or nchunk % 8 != 0
    _dmax = max(ept * (NS - 1) + (nchunk - nchunk % 8) * 128 - (E - SCW), 0)
    assert _dmax % 16 == 0 and _dmax + (nchunk % 8) * 128 + tail <= SCW

    scratch = [
        pltpu.VMEM((SCW,), _I32),
        pltpu.VMEM((SCW,), _I32),
        pltpu.VMEM((128,), _I32),
        pltpu.VMEM((128,), _I32),
        pltpu.VMEM((128,), _I32),
        pltpu.VMEM((128,), _I32),
        pltpu.VMEM((128, 128), _F32),
        pltpu.VMEM((128, 128), _F32),
        pltpu.VMEM((WB, 128), _F32),
        pltpu.VMEM_SHARED((n_nodes, 128), _F32),
        pltpu.SemaphoreType.DMA,
        pltpu.SemaphoreType.DMA,
    ]
    if tail:
        scratch += [
            pltpu.VMEM((tail,), _I32),
            pltpu.VMEM((tail,), _I32),
            pltpu.VMEM((tail, 128), _F32),
        ]

    @functools.partial(
        pl.kernel,
        mesh=mesh,
        out_type=jax.ShapeDtypeStruct((2, n_nodes, 128), _F32),
        scratch_types=scratch,
    )
    def k(table_hbm, edges_hbm, out_hbm,
          srcS, dstS, idxA, dstA, idxB, dstB, rowsA, rowsB,
          zbuf, acc, semA, semB, *tails):
        c = lax.axis_index("c")
        s = lax.axis_index("s")

        def zrow(r, carry):
            for j in range(8):
                zbuf[r, pl.ds(j * 16, 16)] = jnp.zeros((16,), _F32)
            return carry
        lax.fori_loop(0, WB, zrow, 0)

        row0 = s * RPT

        @pl.when(s < n_wtiles)
        def _zero():
            def zcp(t, carry):
                pltpu.sync_copy(zbuf, acc.at[pl.ds(row0 + t * WB, WB)])
                return carry
            lax.fori_loop(0, nwr, zcp, 0)

        ebase = s * ept
        plsc.subcore_barrier()

        def fill(cidx, idx_v, dst_v):
            b0 = ebase + (cidx - cidx % 8) * 128
            d = jnp.maximum(b0 - (E - SCW), 0)

            @pl.when(cidx % 8 == 0)
            def _stage():
                bs = pl.multiple_of(b0 - d, 8)
                pltpu.sync_copy(edges_hbm.at[pl.ds(bs, SCW)], srcS)
                pltpu.sync_copy(edges_hbm.at[pl.ds(E + bs, SCW)], dstS)
            loc = d + (cidx % 8) * 128
            for j in range(8):
                sv = srcS[pl.ds(loc + j * 16, 16)]
                idx_v[pl.ds(j * 16, 16)] = sv * 2 + c
                dst_v[pl.ds(j * 16, 16)] = dstS[pl.ds(loc + j * 16, 16)]

        fill(jnp.int32(0), idxA, dstA)
        pltpu.async_copy(table_hbm.at[idxA], rowsA, semA)
        fill(jnp.int32(1), idxB, dstB)
        pltpu.async_copy(table_hbm.at[idxB], rowsB, semB)

        def step(i2, carry):
            pltpu.make_async_copy(table_hbm.at[idxA], rowsA, semA).wait()
            pltpu.sync_copy(rowsA, acc.at[dstA], add=True)
            fill(2 * i2 + 2, idxA, dstA)
            pltpu.async_copy(table_hbm.at[idxA], rowsA, semA)
            pltpu.make_async_copy(table_hbm.at[idxB], rowsB, semB).wait()
            pltpu.sync_copy(rowsB, acc.at[dstB], add=True)
            fill(2 * i2 + 3, idxB, dstB)
            pltpu.async_copy(table_hbm.at[idxB], rowsB, semB)
            return carry
        lax.fori_loop(0, nchunk // 2 - 1, step, 0)

        pltpu.make_async_copy(table_hbm.at[idxA], rowsA, semA).wait()
        pltpu.sync_copy(rowsA, acc.at[dstA], add=True)
        pltpu.make_async_copy(table_hbm.at[idxB], rowsB, semB).wait()
        pltpu.sync_copy(rowsB, acc.at[dstB], add=True)

        if tail:
            idx_t, dst_t, rows_t = tails
            b0t = ebase + (nchunk - nchunk % 8) * 128
            off = jnp.maximum(b0t - (E - SCW), 0) + (nchunk % 8) * 128
            for j in range(tail // 16):
                sv = srcS[pl.ds(off + j * 16, 16)]
                idx_t[pl.ds(j * 16, 16)] = sv * 2 + c
                dst_t[pl.ds(j * 16, 16)] = dstS[pl.ds(off + j * 16, 16)]
            pltpu.async_copy(table_hbm.at[idx_t], rows_t, semA).wait()
            pltpu.sync_copy(rows_t, acc.at[dst_t], add=True)

        plsc.subcore_barrier()

        @pl.when(s < n_wtiles)
        def _writeout():
            def wout(t, carry):
                r = row0 + t * WB
                pltpu.sync_copy(acc.at[pl.ds(r, WB)], zbuf)
                pltpu.sync_copy(zbuf, out_hbm.at[c, pl.ds(r, WB)])
                return carry
            lax.fori_loop(0, nwr, wout, 0)

    return k(table2, edges)


def _gin_mlp_tc(x, aggT, Wa, ba, Wb, bb, relu_out, interleave_out):
    N, D = x.shape
    BN = 1000
    assert N % BN == 0
    grid = (N // BN,)
    dn = (((1,), (0,)), ((), ()))

    def body(x_ref, agg_ref, wa_ref, ba_ref, wb_ref, bb_ref, o_ref):
        h = x_ref[...] + jnp.concatenate([agg_ref[0], agg_ref[1]], axis=-1)
        u = lax.dot_general(h, wa_ref[...], dn,
                            preferred_element_type=_F32)
        u = jnp.maximum(u + ba_ref[...], 0.0)
        y = lax.dot_general(u, wb_ref[...], dn,
                            preferred_element_type=_F32)
        y = y + bb_ref[...]
        if relu_out:
            y = jnp.maximum(y, 0.0)
        o_ref[...] = y.reshape(o_ref.shape)

    if interleave_out:
        out_sds = jax.ShapeDtypeStruct((N, 2, 128), _F32)
        out_spec = pl.BlockSpec((BN, 2, 128), lambda i: (i, 0, 0))
    else:
        out_sds = jax.ShapeDtypeStruct((N, D), _F32)
        out_spec = pl.BlockSpec((BN, D), lambda i: (i, 0))

    return pl.pallas_call(
        body,
        grid=grid,
        in_specs=[
            pl.BlockSpec((BN, D), lambda i: (i, 0)),
            pl.BlockSpec((2, BN, 128), lambda i: (0, i, 0)),
            pl.BlockSpec((D, D), lambda i: (0, 0)),
            pl.BlockSpec((1, D), lambda i: (0, 0)),
            pl.BlockSpec((D, D), lambda i: (0, 0)),
            pl.BlockSpec((1, D), lambda i: (0, 0)),
        ],
        out_specs=out_spec,
        out_shape=out_sds,
    )(x, aggT, Wa, ba.reshape(1, D), Wb, bb.reshape(1, D))


def kernel(x, ei, W1a, b1a, W1b, b1b, W2a, b2a, W2b, b2b):
    N, D = x.shape
    assert D == 256
    ei32 = ei.astype(_I32)

    E = ei.shape[1]
    x2 = x.reshape(2 * N, 128)
    eiflat = ei32.reshape(2 * E)
    agg1 = _segment_sum_sc(x2, eiflat, N, E)
    y1i = _gin_mlp_tc(x, agg1, W1a, b1a, W1b, b1b,
                      relu_out=True, interleave_out=True)
    agg2 = _segment_sum_sc(y1i.reshape(2 * N, 128), eiflat, N, E)
    out = _gin_mlp_tc(y1i.reshape(N, D), agg2, W2a, b2a, W2b, b2b,
                      relu_out=False, interleave_out=False)
    return out


# speedup vs baseline: 1.2877x; 1.0339x over previous
import functools

import jax
import jax.numpy as jnp
from jax import lax
from jax.experimental import pallas as pl
from jax.experimental.pallas import tpu as pltpu
from jax.experimental.pallas import tpu_sc as plsc

_F32 = jnp.float32
_I32 = jnp.int32


def _segment_sum_sc(table2, edges, n_nodes, n_edges):
    E = n_edges
    NS = 16
    ept = E // NS
    assert ept * NS == E
    nchunk, tail = divmod(ept, 128)
    assert tail % 16 == 0
    WB = 40
    RPT = 1000
    n_wtiles = n_nodes // RPT
    assert n_wtiles * RPT == n_nodes and n_wtiles <= NS and RPT % WB == 0
    nwr = RPT // WB

    mesh = plsc.VectorSubcoreMesh(core_axis_name="c", subcore_axis_name="s")

    assert nchunk % 2 == 0 and nchunk >= 4
    SCW = 1024
    assert tail == 0 or nchunk % 8 != 0
    _dmax = max(ept * (NS - 1) + (nchunk - nchunk % 8) * 128 - (E - SCW), 0)
    assert _dmax % 16 == 0 and _dmax + (nchunk % 8) * 128 + tail <= SCW

    scratch = [
        pltpu.VMEM((SCW,), _I32),
        pltpu.VMEM((SCW,), _I32),
        pltpu.VMEM((128,), _I32),
        pltpu.VMEM((128,), _I32),
        pltpu.VMEM((128,), _I32),
        pltpu.VMEM((128,), _I32),
        pltpu.VMEM((128, 128), _F32),
        pltpu.VMEM((128, 128), _F32),
        pltpu.VMEM((WB, 128), _F32),
        pltpu.VMEM_SHARED((n_nodes, 128), _F32),
        pltpu.SemaphoreType.DMA,
        pltpu.SemaphoreType.DMA,
    ]
    if tail:
        scratch += [
            pltpu.VMEM((tail,), _I32),
            pltpu.VMEM((tail,), _I32),
            pltpu.VMEM((tail, 128), _F32),
        ]

    @functools.partial(
        pl.kernel,
        mesh=mesh,
        out_type=jax.ShapeDtypeStruct((2, n_nodes, 128), _F32),
        scratch_types=scratch,
    )
    def k(table_hbm, edges_hbm, out_hbm,
          srcS, dstS, idxA, dstA, idxB, dstB, rowsA, rowsB,
          zbuf, acc, semA, semB, *tails):
        c = lax.axis_index("c")
        s = lax.axis_index("s")

        def zrow(r, carry):
            for j in range(8):
                zbuf[r, pl.ds(j * 16, 16)] = jnp.zeros((16,), _F32)
            return carry
        lax.fori_loop(0, WB, zrow, 0)

        row0 = s * RPT

        @pl.when(s < n_wtiles)
        def _zero():
            def zcp(t, carry):
                pltpu.sync_copy(zbuf, acc.at[pl.ds(row0 + t * WB, WB)])
                return carry
            lax.fori_loop(0, nwr, zcp, 0)

        ebase = s * ept
        plsc.subcore_barrier()

        def fill(cidx, idx_v, dst_v):
            b0 = ebase + (cidx - cidx % 8) * 128
            d = jnp.maximum(b0 - (E - SCW), 0)

            @pl.when(cidx % 8 == 0)
            def _stage():
                bs = pl.multiple_of(b0 - d, 8)
                pltpu.sync_copy(edges_hbm.at[pl.ds(bs, SCW)], srcS)
                pltpu.sync_copy(edges_hbm.at[pl.ds(E + bs, SCW)], dstS)
            loc = d + (cidx % 8) * 128
            for j in range(8):
                sv = srcS[pl.ds(loc + j * 16, 16)]
                idx_v[pl.ds(j * 16, 16)] = sv * 2 + c
                dst_v[pl.ds(j * 16, 16)] = dstS[pl.ds(loc + j * 16, 16)]

        fill(jnp.int32(0), idxA, dstA)
        pltpu.async_copy(table_hbm.at[idxA], rowsA, semA)
        fill(jnp.int32(1), idxB, dstB)
        pltpu.async_copy(table_hbm.at[idxB], rowsB, semB)

        def step(i2, carry):
            pltpu.make_async_copy(table_hbm.at[idxA], rowsA, semA).wait()
            pltpu.sync_copy(rowsA, acc.at[dstA], add=True)
            fill(2 * i2 + 2, idxA, dstA)
            pltpu.async_copy(table_hbm.at[idxA], rowsA, semA)
            pltpu.make_async_copy(table_hbm.at[idxB], rowsB, semB).wait()
            pltpu.sync_copy(rowsB, acc.at[dstB], add=True)
            fill(2 * i2 + 3, idxB, dstB)
            pltpu.async_copy(table_hbm.at[idxB], rowsB, semB)
            return carry
        lax.fori_loop(0, nchunk // 2 - 1, step, 0)

        pltpu.make_async_copy(table_hbm.at[idxA], rowsA, semA).wait()
        pltpu.sync_copy(rowsA, acc.at[dstA], add=True)
        pltpu.make_async_copy(table_hbm.at[idxB], rowsB, semB).wait()
        pltpu.sync_copy(rowsB, acc.at[dstB], add=True)

        if tail:
            idx_t, dst_t, rows_t = tails
            b0t = ebase + (nchunk - nchunk % 8) * 128
            off = jnp.maximum(b0t - (E - SCW), 0) + (nchunk % 8) * 128
            for j in range(tail // 16):
                sv = srcS[pl.ds(off + j * 16, 16)]
                idx_t[pl.ds(j * 16, 16)] = sv * 2 + c
                dst_t[pl.ds(j * 16, 16)] = dstS[pl.ds(off + j * 16, 16)]
            pltpu.async_copy(table_hbm.at[idx_t], rows_t, semA).wait()
            pltpu.sync_copy(rows_t, acc.at[dst_t], add=True)

        plsc.subcore_barrier()

        @pl.when(s < n_wtiles)
        def _writeout():
            def wout(t, carry):
                r = row0 + t * WB
                pltpu.sync_copy(acc.at[pl.ds(r, WB)], zbuf)
                pltpu.sync_copy(zbuf, out_hbm.at[c, pl.ds(r, WB)])
                return carry
            lax.fori_loop(0, nwr, wout, 0)

    return k(table2, edges)


def _gin_mlp_tc(x, aggT, Wa, ba, Wb, bb, relu_out, x_is_table, out_table):
    if x_is_table:
        N2 = x.shape[0]
        N, D = N2 // 2, 256
    else:
        N, D = x.shape
    BN = 1000
    assert N % BN == 0
    grid = (N // BN,)
    dn = (((1,), (0,)), ((), ()))

    def body(x_ref, agg_ref, wa_ref, ba_ref, wb_ref, bb_ref, o_ref):
        if x_is_table:
            xb = x_ref[...].reshape(BN, 2, 128)
            h = jnp.concatenate([xb[:, 0, :] + agg_ref[0],
                                 xb[:, 1, :] + agg_ref[1]], axis=-1)
        else:
            h = x_ref[...] + jnp.concatenate([agg_ref[0], agg_ref[1]],
                                             axis=-1)
        u = lax.dot_general(h, wa_ref[...], dn,
                            preferred_element_type=_F32)
        u = jnp.maximum(u + ba_ref[...], 0.0)
        y = lax.dot_general(u, wb_ref[...], dn,
                            preferred_element_type=_F32)
        y = y + bb_ref[...]
        if relu_out:
            y = jnp.maximum(y, 0.0)
        o_ref[...] = y.reshape(o_ref.shape)

    if x_is_table:
        x_spec = pl.BlockSpec((2 * BN, 128), lambda i: (i, 0))
    else:
        x_spec = pl.BlockSpec((BN, D), lambda i: (i, 0))
    if out_table:
        out_sds = jax.ShapeDtypeStruct((2 * N, 128), _F32)
        out_spec = pl.BlockSpec((2 * BN, 128), lambda i: (i, 0))
    else:
        out_sds = jax.ShapeDtypeStruct((N, D), _F32)
        out_spec = pl.BlockSpec((BN, D), lambda i: (i, 0))

    return pl.pallas_call(
        body,
        grid=grid,
        in_specs=[
            x_spec,
            pl.BlockSpec((2, BN, 128), lambda i: (0, i, 0)),
            pl.BlockSpec((D, D), lambda i: (0, 0)),
            pl.BlockSpec((1, D), lambda i: (0, 0)),
            pl.BlockSpec((D, D), lambda i: (0, 0)),
            pl.BlockSpec((1, D), lambda i: (0, 0)),
        ],
        out_specs=out_spec,
        out_shape=out_sds,
    )(x, aggT, Wa, ba.reshape(1, D), Wb, bb.reshape(1, D))


def kernel(x, ei, W1a, b1a, W1b, b1b, W2a, b2a, W2b, b2b):
    N, D = x.shape
    assert D == 256
    ei32 = ei.astype(_I32)

    E = ei.shape[1]
    x2 = x.reshape(2 * N, 128)
    eiflat = ei32.reshape(2 * E)
    agg1 = _segment_sum_sc(x2, eiflat, N, E)
    y1t = _gin_mlp_tc(x, agg1, W1a, b1a, W1b, b1b,
                      relu_out=True, x_is_table=False, out_table=True)
    agg2 = _segment_sum_sc(y1t, eiflat, N, E)
    out = _gin_mlp_tc(y1t, agg2, W2a, b2a, W2b, b2b,
                      relu_out=False, x_is_table=True, out_table=False)
    return out


# speedup vs baseline: 1.3526x; 1.0504x over previous
import functools

import jax
import jax.numpy as jnp
from jax import lax
from jax.experimental import pallas as pl
from jax.experimental.pallas import tpu as pltpu
from jax.experimental.pallas import tpu_sc as plsc

_F32 = jnp.float32
_I32 = jnp.int32


def _segment_sum_sc(table2, edges, n_nodes, n_edges):
    E = n_edges
    NS = 16
    ept = E // NS
    assert ept * NS == E
    nchunk, tail = divmod(ept, 128)
    assert tail % 16 == 0
    WB = 40
    RPT = 1000
    n_wtiles = n_nodes // RPT
    assert n_wtiles * RPT == n_nodes and n_wtiles <= NS and RPT % WB == 0
    nwr = RPT // WB

    mesh = plsc.VectorSubcoreMesh(core_axis_name="c", subcore_axis_name="s")

    assert nchunk % 2 == 0 and nchunk >= 4
    SCW = 1024
    assert tail == 0 or nchunk % 8 != 0
    _dmax = max(ept * (NS - 1) + (nchunk - nchunk % 8) * 128 - (E - SCW), 0)
    assert _dmax % 16 == 0 and _dmax + (nchunk % 8) * 128 + tail <= SCW

    scratch = [
        pltpu.VMEM((SCW,), _I32),
        pltpu.VMEM((SCW,), _I32),
        pltpu.VMEM((128,), _I32),
        pltpu.VMEM((128,), _I32),
        pltpu.VMEM((128,), _I32),
        pltpu.VMEM((128,), _I32),
        pltpu.VMEM((128, 128), _F32),
        pltpu.VMEM((128, 128), _F32),
        pltpu.VMEM((WB, 128), _F32),
        pltpu.VMEM_SHARED((n_nodes, 128), _F32),
        pltpu.SemaphoreType.DMA,
        pltpu.SemaphoreType.DMA,
        pltpu.SemaphoreType.DMA,
    ]
    if tail:
        scratch += [
            pltpu.VMEM((tail,), _I32),
            pltpu.VMEM((tail,), _I32),
            pltpu.VMEM((tail, 128), _F32),
        ]

    @functools.partial(
        pl.kernel,
        mesh=mesh,
        out_type=jax.ShapeDtypeStruct((2, n_nodes, 128), _F32),
        scratch_types=scratch,
    )
    def k(table_hbm, edges_hbm, out_hbm,
          srcS, dstS, idxA, dstA, idxB, dstB, rowsA, rowsB,
          zbuf, acc, semA, semB, semZ, *tails):
        c = lax.axis_index("c")
        s = lax.axis_index("s")

        def zrow(r, carry):
            for j in range(8):
                zbuf[r, pl.ds(j * 16, 16)] = jnp.zeros((16,), _F32)
            return carry
        lax.fori_loop(0, WB, zrow, 0)

        row0 = s * RPT

        @pl.when(s < n_wtiles)
        def _zero():
            def zcp(t, carry):
                pltpu.async_copy(zbuf, acc.at[pl.ds(row0 + t * WB, WB)],
                                 semZ)
                return carry
            lax.fori_loop(0, nwr, zcp, 0)

            def zdr(t, carry):
                pltpu.make_async_copy(
                    zbuf, acc.at[pl.ds(row0 + t * WB, WB)], semZ).wait()
                return carry
            lax.fori_loop(0, nwr, zdr, 0)

        ebase = s * ept
        plsc.subcore_barrier()

        def fill(cidx, idx_v, dst_v):
            b0 = ebase + (cidx - cidx % 8) * 128
            d = jnp.maximum(b0 - (E - SCW), 0)

            @pl.when(cidx % 8 == 0)
            def _stage():
                bs = pl.multiple_of(b0 - d, 8)
                pltpu.sync_copy(edges_hbm.at[pl.ds(bs, SCW)], srcS)
                pltpu.sync_copy(edges_hbm.at[pl.ds(E + bs, SCW)], dstS)
            loc = d + (cidx % 8) * 128
            for j in range(8):
                sv = srcS[pl.ds(loc + j * 16, 16)]
                idx_v[pl.ds(j * 16, 16)] = sv * 2 + c
                dst_v[pl.ds(j * 16, 16)] = dstS[pl.ds(loc + j * 16, 16)]

        fill(jnp.int32(0), idxA, dstA)
        pltpu.async_copy(table_hbm.at[idxA], rowsA, semA)
        fill(jnp.int32(1), idxB, dstB)
        pltpu.async_copy(table_hbm.at[idxB], rowsB, semB)

        def step(i2, carry):
            pltpu.make_async_copy(table_hbm.at[idxA], rowsA, semA).wait()
            pltpu.sync_copy(rowsA, acc.at[dstA], add=True)
            fill(2 * i2 + 2, idxA, dstA)
            pltpu.async_copy(table_hbm.at[idxA], rowsA, semA)
            pltpu.make_async_copy(table_hbm.at[idxB], rowsB, semB).wait()
            pltpu.sync_copy(rowsB, acc.at[dstB], add=True)
            fill(2 * i2 + 3, idxB, dstB)
            pltpu.async_copy(table_hbm.at[idxB], rowsB, semB)
            return carry
        lax.fori_loop(0, nchunk // 2 - 1, step, 0)

        pltpu.make_async_copy(table_hbm.at[idxA], rowsA, semA).wait()
        pltpu.sync_copy(rowsA, acc.at[dstA], add=True)
        pltpu.make_async_copy(table_hbm.at[idxB], rowsB, semB).wait()
        pltpu.sync_copy(rowsB, acc.at[dstB], add=True)

        if tail:
            idx_t, dst_t, rows_t = tails
            b0t = ebase + (nchunk - nchunk % 8) * 128
            off = jnp.maximum(b0t - (E - SCW), 0) + (nchunk % 8) * 128
            for j in range(tail // 16):
                sv = srcS[pl.ds(off + j * 16, 16)]
                idx_t[pl.ds(j * 16, 16)] = sv * 2 + c
                dst_t[pl.ds(j * 16, 16)] = dstS[pl.ds(off + j * 16, 16)]
            pltpu.async_copy(table_hbm.at[idx_t], rows_t, semA).wait()
            pltpu.sync_copy(rows_t, acc.at[dst_t], add=True)

        plsc.subcore_barrier()

        @pl.when(s < n_wtiles)
        def _writeout():
            pltpu.sync_copy(acc.at[pl.ds(row0, RPT)],
                            out_hbm.at[c, pl.ds(row0, RPT)])

    return k(table2, edges)


def _gin_mlp_tc(x, aggT, Wa, ba, Wb, bb, relu_out, x_is_table, out_table):
    if x_is_table:
        N2 = x.shape[0]
        N, D = N2 // 2, 256
    else:
        N, D = x.shape
    BN = 1000
    assert N % BN == 0
    grid = (N // BN,)
    dn = (((1,), (0,)), ((), ()))

    def body(x_ref, agg_ref, wa_ref, ba_ref, wb_ref, bb_ref, o_ref):
        if x_is_table:
            xb = x_ref[...].reshape(BN, 2, 128)
            h = jnp.concatenate([xb[:, 0, :] + agg_ref[0],
                                 xb[:, 1, :] + agg_ref[1]], axis=-1)
        else:
            h = x_ref[...] + jnp.concatenate([agg_ref[0], agg_ref[1]],
                                             axis=-1)
        u = lax.dot_general(h, wa_ref[...], dn,
                            preferred_element_type=_F32)
        u = jnp.maximum(u + ba_ref[...], 0.0)
        y = lax.dot_general(u, wb_ref[...], dn,
                            preferred_element_type=_F32)
        y = y + bb_ref[...]
        if relu_out:
            y = jnp.maximum(y, 0.0)
        o_ref[...] = y.reshape(o_ref.shape)

    if x_is_table:
        x_spec = pl.BlockSpec((2 * BN, 128), lambda i: (i, 0))
    else:
        x_spec = pl.BlockSpec((BN, D), lambda i: (i, 0))
    if out_table:
        out_sds = jax.ShapeDtypeStruct((2 * N, 128), _F32)
        out_spec = pl.BlockSpec((2 * BN, 128), lambda i: (i, 0))
    else:
        out_sds = jax.ShapeDtypeStruct((N, D), _F32)
        out_spec = pl.BlockSpec((BN, D), lambda i: (i, 0))

    return pl.pallas_call(
        body,
        grid=grid,
        in_specs=[
            x_spec,
            pl.BlockSpec((2, BN, 128), lambda i: (0, i, 0)),
            pl.BlockSpec((D, D), lambda i: (0, 0)),
            pl.BlockSpec((1, D), lambda i: (0, 0)),
            pl.BlockSpec((D, D), lambda i: (0, 0)),
            pl.BlockSpec((1, D), lambda i: (0, 0)),
        ],
        out_specs=out_spec,
        out_shape=out_sds,
    )(x, aggT, Wa, ba.reshape(1, D), Wb, bb.reshape(1, D))


def kernel(x, ei, W1a, b1a, W1b, b1b, W2a, b2a, W2b, b2b):
    N, D = x.shape
    assert D == 256
    ei32 = ei.astype(_I32)

    E = ei.shape[1]
    x2 = x.reshape(2 * N, 128)
    eiflat = ei32.reshape(2 * E)
    agg1 = _segment_sum_sc(x2, eiflat, N, E)
    y1t = _gin_mlp_tc(x, agg1, W1a, b1a, W1b, b1b,
                      relu_out=True, x_is_table=False, out_table=True)
    agg2 = _segment_sum_sc(y1t, eiflat, N, E)
    out = _gin_mlp_tc(y1t, agg2, W2a, b2a, W2b, b2b,
                      relu_out=False, x_is_table=True, out_table=False)
    return out
